# baseline (speedup 1.0000x reference)
import functools

import jax
import jax.numpy as jnp
from jax import lax
from jax.experimental import pallas as pl
from jax.experimental.pallas import tpu as pltpu
from jax.experimental.pallas import tpu_sc as plsc

N = 10000
E = 160000
LATENT = 256
HIDDEN = 256
OUT = 128

NTILES = 16
PT = E // NTILES
CH = 80
NCH = PT // CH
NCHP = 64
TRA = 64
EP = 32 * NCHP * CH
NPAD = N + 8
RB = 1000
RPT = 640
RPT_LAST = N - 15 * RPT


def _sc_mesh():
    return plsc.VectorSubcoreMesh(core_axis_name="c", subcore_axis_name="s")


def _copy_share(sid, src, dst):
    @pl.when(sid < 15)
    def _():
        off = pl.multiple_of(sid * RPT, 8)
        pltpu.sync_copy(src.at[pl.ds(off, RPT)], dst.at[pl.ds(off, RPT)])

    @pl.when(sid == 15)
    def _():
        pltpu.sync_copy(src.at[pl.ds(15 * RPT, RPT_LAST)],
                        dst.at[pl.ds(15 * RPT, RPT_LAST)])


DEGW = 128


@functools.partial(
    pl.kernel,
    out_type=jax.ShapeDtypeStruct((2, N, DEGW), jnp.float32),
    mesh=_sc_mesh(),
    scratch_types=[
        pltpu.VMEM((NCHP, CH), jnp.int32),
        pltpu.VMEM((CH, DEGW), jnp.float32),
        pltpu.VMEM_SHARED((NPAD, DEGW), jnp.float32),
    ],
)
def _deg_kernel(dst_hbm, ones_hbm, zeros_hbm, out_hbm, dstv, ones, acc):
    cid = lax.axis_index("c")
    sid = lax.axis_index("s")
    pltpu.sync_copy(dst_hbm.at[cid * NTILES + sid], dstv)
    pltpu.sync_copy(ones_hbm, ones)
    _copy_share(sid, zeros_hbm, acc)
    plsc.subcore_barrier()

    def count(j, carry):
        pltpu.sync_copy(ones, acc.at[dstv.at[j]], add=True)
        return carry

    lax.fori_loop(0, NCHP, count, 0)
    plsc.subcore_barrier()
    _copy_share(sid, acc, out_hbm.at[cid])


def _gs_pipeline(nch, plane, srcv, dstv, rows_a, rows_b, acc, sem):
    pltpu.async_copy(plane.at[srcv.at[0]], rows_a, sem)

    def pair(i, carry):
        j = 2 * i
        pltpu.make_async_copy(plane.at[srcv.at[j]], rows_a, sem).wait()
        pltpu.async_copy(plane.at[srcv.at[j + 1]], rows_b, sem)
        pltpu.sync_copy(rows_a, acc.at[dstv.at[j]], add=True)
        pltpu.make_async_copy(plane.at[srcv.at[j + 1]], rows_b, sem).wait()

        @pl.when(j + 2 < nch)
        def _():
            pltpu.async_copy(plane.at[srcv.at[j + 2]], rows_a, sem)

        pltpu.sync_copy(rows_b, acc.at[dstv.at[j + 1]], add=True)
        return carry

    lax.fori_loop(0, nch // 2, pair, 0)
    if nch % 2:
        pltpu.make_async_copy(plane.at[srcv.at[nch - 1]], rows_a, sem).wait()
        pltpu.sync_copy(rows_a, acc.at[dstv.at[nch - 1]], add=True)


@functools.partial(
    pl.kernel,
    out_type=jax.ShapeDtypeStruct((2, N, HIDDEN // 2), jnp.float32),
    mesh=_sc_mesh(),
    scratch_types=[
        pltpu.VMEM((TRA, CH), jnp.int32),
        pltpu.VMEM((TRA, CH), jnp.int32),
        pltpu.VMEM((CH, HIDDEN // 2), jnp.float32),
        pltpu.VMEM((CH, HIDDEN // 2), jnp.float32),
        pltpu.VMEM_SHARED((NPAD, HIDDEN // 2), jnp.float32),
        pltpu.SemaphoreType.DMA,
    ],
)
def _agg1_kernel(y_hbm, srca_hbm, srcb_hbm, dsta_hbm, dstb_hbm, zeros_hbm,
                 out_hbm, srcv, dstv, rows_a, rows_b, acc, sem):
    cid = lax.axis_index("c")
    sid = lax.axis_index("s")
    plane = y_hbm.at[cid]
    pltpu.sync_copy(srca_hbm.at[sid], srcv)
    pltpu.sync_copy(dsta_hbm.at[sid], dstv)
    _copy_share(sid, zeros_hbm, acc)
    plsc.subcore_barrier()
    _gs_pipeline(TRA, plane, srcv, dstv, rows_a, rows_b, acc, sem)
    pltpu.sync_copy(srcb_hbm.at[sid], srcv)
    pltpu.sync_copy(dstb_hbm.at[sid], dstv)
    _gs_pipeline(TRA, plane, srcv, dstv, rows_a, rows_b, acc, sem)
    plsc.subcore_barrier()
    _copy_share(sid, acc, out_hbm.at[cid])


@functools.partial(
    pl.kernel,
    out_type=jax.ShapeDtypeStruct((2, N, OUT), jnp.float32),
    mesh=_sc_mesh(),
    scratch_types=[
        pltpu.VMEM((NCHP, CH), jnp.int32),
        pltpu.VMEM((NCHP, CH), jnp.int32),
        pltpu.VMEM((CH, OUT), jnp.float32),
        pltpu.VMEM((CH, OUT), jnp.float32),
        pltpu.VMEM_SHARED((NPAD, OUT), jnp.float32),
        pltpu.SemaphoreType.DMA,
    ],
)
def _agg2_kernel(y_hbm, src_hbm, dst_hbm, zeros_hbm, out_hbm,
                 srcv, dstv, rows_a, rows_b, acc, sem):
    cid = lax.axis_index("c")
    sid = lax.axis_index("s")
    w = cid * NTILES + sid
    pltpu.sync_copy(src_hbm.at[w], srcv)
    pltpu.sync_copy(dst_hbm.at[w], dstv)
    _copy_share(sid, zeros_hbm, acc)
    plsc.subcore_barrier()
    _gs_pipeline(NCHP, y_hbm, srcv, dstv, rows_a, rows_b, acc, sem)
    plsc.subcore_barrier()
    _copy_share(sid, acc, out_hbm.at[cid])


def _b_body(x_ref, w_ref, deg_ref, y_ref, dinv_ref):
    total = deg_ref[0, :, 0:1] + deg_ref[1, :, 0:1]
    dcol = lax.rsqrt(total + 1.0)
    dinv_ref[...] = dcol
    y = dcol * jnp.dot(x_ref[...], w_ref[...],
                       preferred_element_type=jnp.float32)
    y_ref[0] = y[:, :HIDDEN // 2]
    y_ref[1] = y[:, HIDDEN // 2:]


def _d_body(agg_ref, y1_ref, dinv_ref, b1_ref, w2_ref, y2_ref):
    dinv = dinv_ref[...]
    s = jnp.concatenate([agg_ref[0] + y1_ref[0], agg_ref[1] + y1_ref[1]],
                        axis=1)
    h = jnp.maximum(dinv * s + b1_ref[...], 0.0)
    y2_ref[...] = dinv * jnp.dot(h, w2_ref[...],
                                 preferred_element_type=jnp.float32)


def _f_body(agg_ref, y2_ref, dinv_ref, b2_ref, out_ref):
    s = agg_ref[0] + agg_ref[1] + y2_ref[...]
    out_ref[...] = dinv_ref[...] * s + b2_ref[...]


def _tc_b(x, w1, degp):
    return pl.pallas_call(
        _b_body,
        grid=(N // RB,),
        in_specs=[
            pl.BlockSpec((RB, LATENT), lambda i: (i, 0)),
            pl.BlockSpec((LATENT, HIDDEN), lambda i: (0, 0)),
            pl.BlockSpec((2, RB, DEGW), lambda i: (0, i, 0)),
        ],
        out_specs=[
            pl.BlockSpec((2, RB, HIDDEN // 2), lambda i: (0, i, 0)),
            pl.BlockSpec((RB, 1), lambda i: (i, 0)),
        ],
        out_shape=[
            jax.ShapeDtypeStruct((2, N, HIDDEN // 2), jnp.float32),
            jax.ShapeDtypeStruct((N, 1), jnp.float32),
        ],
    )(x, w1, degp)


def _tc_d(agg1, y1, dinv, b1, w2):
    return pl.pallas_call(
        _d_body,
        grid=(N // RB,),
        in_specs=[
            pl.BlockSpec((2, RB, HIDDEN // 2), lambda i: (0, i, 0)),
            pl.BlockSpec((2, RB, HIDDEN // 2), lambda i: (0, i, 0)),
            pl.BlockSpec((RB, 1), lambda i: (i, 0)),
            pl.BlockSpec((1, HIDDEN), lambda i: (0, 0)),
            pl.BlockSpec((HIDDEN, OUT), lambda i: (0, 0)),
        ],
        out_specs=pl.BlockSpec((RB, OUT), lambda i: (i, 0)),
        out_shape=jax.ShapeDtypeStruct((N, OUT), jnp.float32),
    )(agg1, y1, dinv, b1, w2)


def _tc_f(agg2, y2, dinv, b2):
    return pl.pallas_call(
        _f_body,
        grid=(N // RB,),
        in_specs=[
            pl.BlockSpec((2, RB, OUT), lambda i: (0, i, 0)),
            pl.BlockSpec((RB, OUT), lambda i: (i, 0)),
            pl.BlockSpec((RB, 1), lambda i: (i, 0)),
            pl.BlockSpec((1, OUT), lambda i: (0, 0)),
        ],
        out_specs=pl.BlockSpec((RB, OUT), lambda i: (i, 0)),
        out_shape=jax.ShapeDtypeStruct((N, OUT), jnp.float32),
    )(agg2, y2, dinv, b2)


def kernel(x, edge_index, W1, b1, W2, b2):
    src = edge_index[0].astype(jnp.int32)
    dst = edge_index[1].astype(jnp.int32)
    pad = EP - E
    srcf = jnp.concatenate([src, jnp.zeros((pad,), jnp.int32)])
    dstf = jnp.concatenate([dst, jnp.full((pad,), N, jnp.int32)])
    srcp = srcf.reshape(32, NCHP, CH)
    dstp = dstf.reshape(32, NCHP, CH)
    src1t = srcf.reshape(NTILES, 2 * TRA, CH)
    dst1t = dstf.reshape(NTILES, 2 * TRA, CH)
    src1a, src1b = src1t[:, :TRA], src1t[:, TRA:]
    dst1a, dst1b = dst1t[:, :TRA], dst1t[:, TRA:]

    zeros_h = jnp.zeros((N, HIDDEN // 2), jnp.float32)
    zeros_o = jnp.zeros((N, OUT), jnp.float32)
    zeros_d = jnp.zeros((N, DEGW), jnp.float32)
    ones_d = jnp.ones((CH, DEGW), jnp.float32)

    degp = _deg_kernel(dstp, ones_d, zeros_d)
    y1, dinv = _tc_b(x, W1, degp)
    agg1 = _agg1_kernel(y1, src1a, src1b, dst1a, dst1b, zeros_h)
    y2 = _tc_d(agg1, y1, dinv, b1.reshape(1, HIDDEN), W2)
    agg2 = _agg2_kernel(y2, srcp, dstp, zeros_o)
    return _tc_f(agg2, y2, dinv, b2.reshape(1, OUT))


# speedup vs baseline: 1.1144x; 1.1144x over previous
import functools

import jax
import jax.numpy as jnp
from jax import lax
from jax.experimental import pallas as pl
from jax.experimental.pallas import tpu as pltpu
from jax.experimental.pallas import tpu_sc as plsc

N = 10000
E = 160000
LATENT = 256
HIDDEN = 256
OUT = 128

NTILES = 16
PT = E // NTILES
CH = 80
NCH = PT // CH
NCHP = 64
TRA = 64
EP = 32 * NCHP * CH
NPAD = N + 8
RB = 1000
RPT = 640
RPT_LAST = N - 15 * RPT


def _sc_mesh():
    return plsc.VectorSubcoreMesh(core_axis_name="c", subcore_axis_name="s")


def _copy_share(sid, src, dst):
    @pl.when(sid < 15)
    def _():
        off = pl.multiple_of(sid * RPT, 8)
        pltpu.sync_copy(src.at[pl.ds(off, RPT)], dst.at[pl.ds(off, RPT)])

    @pl.when(sid == 15)
    def _():
        pltpu.sync_copy(src.at[pl.ds(15 * RPT, RPT_LAST)],
                        dst.at[pl.ds(15 * RPT, RPT_LAST)])


DEGW = 128


@functools.partial(
    pl.kernel,
    out_type=jax.ShapeDtypeStruct((2, N, DEGW), jnp.float32),
    mesh=_sc_mesh(),
    scratch_types=[
        pltpu.VMEM((NCHP, CH), jnp.int32),
        pltpu.VMEM((CH, DEGW), jnp.float32),
        pltpu.VMEM_SHARED((NPAD, DEGW), jnp.float32),
    ],
)
def _deg_kernel(dst_hbm, ones_hbm, zeros_hbm, out_hbm, dstv, ones, acc):
    cid = lax.axis_index("c")
    sid = lax.axis_index("s")
    pltpu.sync_copy(dst_hbm.at[cid * NTILES + sid], dstv)
    pltpu.sync_copy(ones_hbm, ones)
    _copy_share(sid, zeros_hbm, acc)
    plsc.subcore_barrier()

    def count(j, carry):
        pltpu.sync_copy(ones, acc.at[dstv.at[j]], add=True)
        return carry

    lax.fori_loop(0, NCHP, count, 0)
    plsc.subcore_barrier()
    _copy_share(sid, acc, out_hbm.at[cid])


def _gs_serial(nch, plane, srcv, dstv, rows, acc, sem):
    def chunk(j, carry):
        pltpu.async_copy(plane.at[srcv.at[j]], rows, sem).wait()
        pltpu.sync_copy(rows, acc.at[dstv.at[j]], add=True)
        return carry

    lax.fori_loop(0, nch, chunk, 0)


@functools.partial(
    pl.kernel,
    out_type=jax.ShapeDtypeStruct((2, N, HIDDEN // 2), jnp.float32),
    mesh=_sc_mesh(),
    scratch_types=[
        pltpu.VMEM((NCH, CH), jnp.int32),
        pltpu.VMEM((NCH, CH), jnp.int32),
        pltpu.VMEM((CH, HIDDEN // 2), jnp.float32),
        pltpu.VMEM_SHARED((N, HIDDEN // 2), jnp.float32),
        pltpu.SemaphoreType.DMA,
    ],
)
def _agg1_kernel(y_hbm, src_hbm, dst_hbm, zeros_hbm, out_hbm,
                 srcv, dstv, rows, acc, sem):
    cid = lax.axis_index("c")
    sid = lax.axis_index("s")
    pltpu.sync_copy(src_hbm.at[sid], srcv)
    pltpu.sync_copy(dst_hbm.at[sid], dstv)
    _copy_share(sid, zeros_hbm, acc)
    plsc.subcore_barrier()
    _gs_serial(NCH, y_hbm.at[cid], srcv, dstv, rows, acc, sem)
    plsc.subcore_barrier()
    _copy_share(sid, acc, out_hbm.at[cid])


@functools.partial(
    pl.kernel,
    out_type=jax.ShapeDtypeStruct((2, N, OUT), jnp.float32),
    mesh=_sc_mesh(),
    scratch_types=[
        pltpu.VMEM((NCHP, CH), jnp.int32),
        pltpu.VMEM((NCHP, CH), jnp.int32),
        pltpu.VMEM((CH, OUT), jnp.float32),
        pltpu.VMEM_SHARED((NPAD, OUT), jnp.float32),
        pltpu.SemaphoreType.DMA,
    ],
)
def _agg2_kernel(y_hbm, src_hbm, dst_hbm, zeros_hbm, out_hbm,
                 srcv, dstv, rows, acc, sem):
    cid = lax.axis_index("c")
    sid = lax.axis_index("s")
    w = cid * NTILES + sid
    pltpu.sync_copy(src_hbm.at[w], srcv)
    pltpu.sync_copy(dst_hbm.at[w], dstv)
    _copy_share(sid, zeros_hbm, acc)
    plsc.subcore_barrier()
    _gs_serial(NCHP, y_hbm, srcv, dstv, rows, acc, sem)
    plsc.subcore_barrier()
    _copy_share(sid, acc, out_hbm.at[cid])


def _b_body(x_ref, w_ref, deg_ref, y_ref, dinv_ref):
    total = deg_ref[0, :, 0:1] + deg_ref[1, :, 0:1]
    dcol = lax.rsqrt(total + 1.0)
    dinv_ref[...] = dcol
    y = dcol * jnp.dot(x_ref[...], w_ref[...],
                       preferred_element_type=jnp.float32)
    y_ref[0] = y[:, :HIDDEN // 2]
    y_ref[1] = y[:, HIDDEN // 2:]


def _d_body(agg_ref, y1_ref, dinv_ref, b1_ref, w2_ref, y2_ref):
    dinv = dinv_ref[...]
    s = jnp.concatenate([agg_ref[0] + y1_ref[0], agg_ref[1] + y1_ref[1]],
                        axis=1)
    h = jnp.maximum(dinv * s + b1_ref[...], 0.0)
    y2_ref[...] = dinv * jnp.dot(h, w2_ref[...],
                                 preferred_element_type=jnp.float32)


def _f_body(agg_ref, y2_ref, dinv_ref, b2_ref, out_ref):
    s = agg_ref[0] + agg_ref[1] + y2_ref[...]
    out_ref[...] = dinv_ref[...] * s + b2_ref[...]


def _tc_b(x, w1, degp):
    return pl.pallas_call(
        _b_body,
        grid=(N // RB,),
        in_specs=[
            pl.BlockSpec((RB, LATENT), lambda i: (i, 0)),
            pl.BlockSpec((LATENT, HIDDEN), lambda i: (0, 0)),
            pl.BlockSpec((2, RB, DEGW), lambda i: (0, i, 0)),
        ],
        out_specs=[
            pl.BlockSpec((2, RB, HIDDEN // 2), lambda i: (0, i, 0)),
            pl.BlockSpec((RB, 1), lambda i: (i, 0)),
        ],
        out_shape=[
            jax.ShapeDtypeStruct((2, N, HIDDEN // 2), jnp.float32),
            jax.ShapeDtypeStruct((N, 1), jnp.float32),
        ],
    )(x, w1, degp)


def _tc_d(agg1, y1, dinv, b1, w2):
    return pl.pallas_call(
        _d_body,
        grid=(N // RB,),
        in_specs=[
            pl.BlockSpec((2, RB, HIDDEN // 2), lambda i: (0, i, 0)),
            pl.BlockSpec((2, RB, HIDDEN // 2), lambda i: (0, i, 0)),
            pl.BlockSpec((RB, 1), lambda i: (i, 0)),
            pl.BlockSpec((1, HIDDEN), lambda i: (0, 0)),
            pl.BlockSpec((HIDDEN, OUT), lambda i: (0, 0)),
        ],
        out_specs=pl.BlockSpec((RB, OUT), lambda i: (i, 0)),
        out_shape=jax.ShapeDtypeStruct((N, OUT), jnp.float32),
    )(agg1, y1, dinv, b1, w2)


def _tc_f(agg2, y2, dinv, b2):
    return pl.pallas_call(
        _f_body,
        grid=(N // RB,),
        in_specs=[
            pl.BlockSpec((2, RB, OUT), lambda i: (0, i, 0)),
            pl.BlockSpec((RB, OUT), lambda i: (i, 0)),
            pl.BlockSpec((RB, 1), lambda i: (i, 0)),
            pl.BlockSpec((1, OUT), lambda i: (0, 0)),
        ],
        out_specs=pl.BlockSpec((RB, OUT), lambda i: (i, 0)),
        out_shape=jax.ShapeDtypeStruct((N, OUT), jnp.float32),
    )(agg2, y2, dinv, b2)


def kernel(x, edge_index, W1, b1, W2, b2):
    src = edge_index[0].astype(jnp.int32)
    dst = edge_index[1].astype(jnp.int32)
    src1 = src.reshape(NTILES, NCH, CH)
    dst1 = dst.reshape(NTILES, NCH, CH)
    pad = EP - E
    srcp = jnp.concatenate([src, jnp.zeros((pad,), jnp.int32)])
    dstp = jnp.concatenate([dst, jnp.full((pad,), N, jnp.int32)])
    srcp = srcp.reshape(32, NCHP, CH)
    dstp = dstp.reshape(32, NCHP, CH)

    zeros_h = jnp.zeros((N, HIDDEN // 2), jnp.float32)
    zeros_o = jnp.zeros((N, OUT), jnp.float32)
    zeros_d = jnp.zeros((N, DEGW), jnp.float32)
    ones_d = jnp.ones((CH, DEGW), jnp.float32)

    degp = _deg_kernel(dstp, ones_d, zeros_d)
    y1, dinv = _tc_b(x, W1, degp)
    agg1 = _agg1_kernel(y1, src1, dst1, zeros_h)
    y2 = _tc_d(agg1, y1, dinv, b1.reshape(1, HIDDEN), W2)
    agg2 = _agg2_kernel(y2, srcp, dstp, zeros_o)
    return _tc_f(agg2, y2, dinv, b2.reshape(1, OUT))


# speedup vs baseline: 1.6203x; 1.4539x over previous
import functools

import jax
import jax.numpy as jnp
from jax import lax
from jax.experimental import pallas as pl
from jax.experimental.pallas import tpu as pltpu
from jax.experimental.pallas import tpu_sc as plsc

N = 10000
E = 160000
LATENT = 256
HIDDEN = 256
OUT = 128

NTILES = 16
PT = E // NTILES
CH = 80
NCH = PT // CH
NCHP = 64
TRA = 64
EP = 32 * NCHP * CH
NPAD = N + 8
RB = 1000
RPT = 640
RPT_LAST = N - 15 * RPT


def _sc_mesh():
    return plsc.VectorSubcoreMesh(core_axis_name="c", subcore_axis_name="s")


def _copy_share(sid, src, dst):
    @pl.when(sid < 15)
    def _():
        off = pl.multiple_of(sid * RPT, 8)
        pltpu.sync_copy(src.at[pl.ds(off, RPT)], dst.at[pl.ds(off, RPT)])

    @pl.when(sid == 15)
    def _():
        pltpu.sync_copy(src.at[pl.ds(15 * RPT, RPT_LAST)],
                        dst.at[pl.ds(15 * RPT, RPT_LAST)])


DEGW = 128


@functools.partial(
    pl.kernel,
    out_type=jax.ShapeDtypeStruct((2, N, DEGW), jnp.float32),
    mesh=_sc_mesh(),
    scratch_types=[
        pltpu.VMEM((NCHP, CH), jnp.int32),
        pltpu.VMEM((CH, DEGW), jnp.float32),
        pltpu.VMEM_SHARED((NPAD, DEGW), jnp.float32),
    ],
)
def _deg_kernel(dst_hbm, ones_hbm, zeros_hbm, out_hbm, dstv, ones, acc):
    cid = lax.axis_index("c")
    sid = lax.axis_index("s")
    pltpu.sync_copy(dst_hbm.at[cid * NTILES + sid], dstv)
    pltpu.sync_copy(ones_hbm, ones)
    _copy_share(sid, zeros_hbm, acc)
    plsc.subcore_barrier()

    def count(j, carry):
        pltpu.sync_copy(ones, acc.at[dstv.at[j]], add=True)
        return carry

    lax.fori_loop(0, NCHP, count, 0)
    plsc.subcore_barrier()
    _copy_share(sid, acc, out_hbm.at[cid])


def _gs_serial(nch, plane, srcv, dstv, rows, acc, sem):
    def chunk(j, carry):
        pltpu.async_copy(plane.at[srcv.at[j]], rows, sem).wait()
        pltpu.sync_copy(rows, acc.at[dstv.at[j]], add=True)
        return carry

    lax.fori_loop(0, nch, chunk, 0)


@functools.partial(
    pl.kernel,
    out_type=jax.ShapeDtypeStruct((2, N, HIDDEN // 2), jnp.float32),
    mesh=_sc_mesh(),
    scratch_types=[
        pltpu.VMEM((NCH, CH), jnp.int32),
        pltpu.VMEM((NCH, CH), jnp.int32),
        pltpu.VMEM((CH, HIDDEN // 2), jnp.float32),
        pltpu.VMEM_SHARED((N, HIDDEN // 2), jnp.float32),
        pltpu.SemaphoreType.DMA,
    ],
)
def _agg1_kernel(y_hbm, src_hbm, dst_hbm, zeros_hbm, out_hbm,
                 srcv, dstv, rows, acc, sem):
    cid = lax.axis_index("c")
    sid = lax.axis_index("s")
    pltpu.sync_copy(src_hbm.at[sid], srcv)
    pltpu.sync_copy(dst_hbm.at[sid], dstv)
    _copy_share(sid, zeros_hbm, acc)
    plsc.subcore_barrier()
    _gs_serial(NCH, y_hbm.at[cid], srcv, dstv, rows, acc, sem)
    plsc.subcore_barrier()
    _copy_share(sid, acc, out_hbm.at[cid])


@functools.partial(
    pl.kernel,
    out_type=jax.ShapeDtypeStruct((2, N, OUT), jnp.float32),
    mesh=_sc_mesh(),
    scratch_types=[
        pltpu.VMEM((NCHP, CH), jnp.int32),
        pltpu.VMEM((NCHP, CH), jnp.int32),
        pltpu.VMEM((CH, OUT), jnp.float32),
        pltpu.VMEM_SHARED((NPAD, OUT), jnp.float32),
        pltpu.SemaphoreType.DMA,
    ],
)
def _agg2_kernel(y_hbm, src_hbm, dst_hbm, zeros_hbm, out_hbm,
                 srcv, dstv, rows, acc, sem):
    cid = lax.axis_index("c")
    sid = lax.axis_index("s")
    w = cid * NTILES + sid
    pltpu.sync_copy(src_hbm.at[w], srcv)
    pltpu.sync_copy(dst_hbm.at[w], dstv)
    _copy_share(sid, zeros_hbm, acc)
    plsc.subcore_barrier()
    _gs_serial(NCHP, y_hbm, srcv, dstv, rows, acc, sem)
    plsc.subcore_barrier()
    _copy_share(sid, acc, out_hbm.at[cid])


def _b_body(x_ref, w_ref, deg_ref, y_ref, dinv_ref):
    total = deg_ref[0, :, 0:1] + deg_ref[1, :, 0:1]
    dcol = lax.rsqrt(total + 1.0)
    dinv_ref[...] = dcol
    y = dcol * jnp.dot(x_ref[...], w_ref[...],
                       preferred_element_type=jnp.float32)
    y_ref[0] = y[:, :HIDDEN // 2]
    y_ref[1] = y[:, HIDDEN // 2:]


def _d_body(agg_ref, y1_ref, dinv_ref, b1_ref, w2_ref, y2_ref):
    dinv = dinv_ref[...]
    s = jnp.concatenate([agg_ref[0] + y1_ref[0], agg_ref[1] + y1_ref[1]],
                        axis=1)
    h = jnp.maximum(dinv * s + b1_ref[...], 0.0)
    y2_ref[...] = dinv * jnp.dot(h, w2_ref[...],
                                 preferred_element_type=jnp.float32)


def _f_body(agg_ref, y2_ref, dinv_ref, b2_ref, out_ref):
    s = agg_ref[0] + agg_ref[1] + y2_ref[...]
    out_ref[...] = dinv_ref[...] * s + b2_ref[...]


def _tc_b(x, w1, degp):
    return pl.pallas_call(
        _b_body,
        grid=(N // RB,),
        in_specs=[
            pl.BlockSpec((RB, LATENT), lambda i: (i, 0)),
            pl.BlockSpec((LATENT, HIDDEN), lambda i: (0, 0)),
            pl.BlockSpec((2, RB, DEGW), lambda i: (0, i, 0)),
        ],
        out_specs=[
            pl.BlockSpec((2, RB, HIDDEN // 2), lambda i: (0, i, 0)),
            pl.BlockSpec((RB, 1), lambda i: (i, 0)),
        ],
        out_shape=[
            jax.ShapeDtypeStruct((2, N, HIDDEN // 2), jnp.float32),
            jax.ShapeDtypeStruct((N, 1), jnp.float32),
        ],
    )(x, w1, degp)


def _tc_d(agg1, y1, dinv, b1, w2):
    return pl.pallas_call(
        _d_body,
        grid=(N // RB,),
        in_specs=[
            pl.BlockSpec((2, RB, HIDDEN // 2), lambda i: (0, i, 0)),
            pl.BlockSpec((2, RB, HIDDEN // 2), lambda i: (0, i, 0)),
            pl.BlockSpec((RB, 1), lambda i: (i, 0)),
            pl.BlockSpec((1, HIDDEN), lambda i: (0, 0)),
            pl.BlockSpec((HIDDEN, OUT), lambda i: (0, 0)),
        ],
        out_specs=pl.BlockSpec((RB, OUT), lambda i: (i, 0)),
        out_shape=jax.ShapeDtypeStruct((N, OUT), jnp.float32),
    )(agg1, y1, dinv, b1, w2)


def _tc_f(agg2, y2, dinv, b2):
    return pl.pallas_call(
        _f_body,
        grid=(N // RB,),
        in_specs=[
            pl.BlockSpec((2, RB, OUT), lambda i: (0, i, 0)),
            pl.BlockSpec((RB, OUT), lambda i: (i, 0)),
            pl.BlockSpec((RB, 1), lambda i: (i, 0)),
            pl.BlockSpec((1, OUT), lambda i: (0, 0)),
        ],
        out_specs=pl.BlockSpec((RB, OUT), lambda i: (i, 0)),
        out_shape=jax.ShapeDtypeStruct((N, OUT), jnp.float32),
    )(agg2, y2, dinv, b2)


def kernel(x, edge_index, W1, b1, W2, b2):
    src = edge_index[0].astype(jnp.int32)
    dst = edge_index[1].astype(jnp.int32)
    src1 = src.reshape(NTILES, NCH, CH)
    dst1 = dst.reshape(NTILES, NCH, CH)
    pad = EP - E
    pad_src = jnp.arange(pad, dtype=jnp.int32) % N
    pad_dst = N + jnp.arange(pad, dtype=jnp.int32) % 8
    srcp = jnp.concatenate([src, pad_src])
    dstp = jnp.concatenate([dst, pad_dst])
    srcp = srcp.reshape(32, NCHP, CH)
    dstp = dstp.reshape(32, NCHP, CH)

    zeros_h = jnp.zeros((N, HIDDEN // 2), jnp.float32)
    zeros_o = jnp.zeros((N, OUT), jnp.float32)
    zeros_d = jnp.zeros((N, DEGW), jnp.float32)
    ones_d = jnp.ones((CH, DEGW), jnp.float32)

    degp = _deg_kernel(dstp, ones_d, zeros_d)
    y1, dinv = _tc_b(x, W1, degp)
    agg1 = _agg1_kernel(y1, src1, dst1, zeros_h)
    y2 = _tc_d(agg1, y1, dinv, b1.reshape(1, HIDDEN), W2)
    agg2 = _agg2_kernel(y2, srcp, dstp, zeros_o)
    return _tc_f(agg2, y2, dinv, b2.reshape(1, OUT))


# speedup vs baseline: 1.8118x; 1.1182x over previous
import functools

import jax
import jax.numpy as jnp
from jax import lax
from jax.experimental import pallas as pl
from jax.experimental.pallas import tpu as pltpu
from jax.experimental.pallas import tpu_sc as plsc

N = 10000
E = 160000
LATENT = 256
HIDDEN = 256
OUT = 128

NTILES = 16
CH = 128
NCHP = 40
TRA = 40
EP = 32 * NCHP * CH
NPAD = N + 8
RB = 1000
RPT = 640
RPT_LAST = N - 15 * RPT


def _sc_mesh():
    return plsc.VectorSubcoreMesh(core_axis_name="c", subcore_axis_name="s")


def _copy_share(sid, src, dst):
    @pl.when(sid < 15)
    def _():
        off = pl.multiple_of(sid * RPT, 8)
        pltpu.sync_copy(src.at[pl.ds(off, RPT)], dst.at[pl.ds(off, RPT)])

    @pl.when(sid == 15)
    def _():
        pltpu.sync_copy(src.at[pl.ds(15 * RPT, RPT_LAST)],
                        dst.at[pl.ds(15 * RPT, RPT_LAST)])


DEGW = 128


@functools.partial(
    pl.kernel,
    out_type=jax.ShapeDtypeStruct((2, N, DEGW), jnp.float32),
    mesh=_sc_mesh(),
    scratch_types=[
        pltpu.VMEM((NCHP, CH), jnp.int32),
        pltpu.VMEM((CH, DEGW), jnp.float32),
        pltpu.VMEM_SHARED((NPAD, DEGW), jnp.float32),
    ],
)
def _deg_kernel(dst_hbm, ones_hbm, zeros_hbm, out_hbm, dstv, ones, acc):
    cid = lax.axis_index("c")
    sid = lax.axis_index("s")
    pltpu.sync_copy(dst_hbm.at[cid * NTILES + sid], dstv)
    pltpu.sync_copy(ones_hbm, ones)
    _copy_share(sid, zeros_hbm, acc)
    plsc.subcore_barrier()

    def count(j, carry):
        pltpu.sync_copy(ones, acc.at[dstv.at[j]], add=True)
        return carry

    lax.fori_loop(0, NCHP, count, 0)
    plsc.subcore_barrier()
    _copy_share(sid, acc, out_hbm.at[cid])


def _gs_serial(nch, plane, srcv, dstv, rows, acc, sem):
    def chunk(j, carry):
        pltpu.async_copy(plane.at[srcv.at[j]], rows, sem).wait()
        pltpu.sync_copy(rows, acc.at[dstv.at[j]], add=True)
        return carry

    lax.fori_loop(0, nch, chunk, 0)


@functools.partial(
    pl.kernel,
    out_type=jax.ShapeDtypeStruct((2, N, HIDDEN // 2), jnp.float32),
    mesh=_sc_mesh(),
    scratch_types=[
        pltpu.VMEM((TRA, CH), jnp.int32),
        pltpu.VMEM((TRA, CH), jnp.int32),
        pltpu.VMEM((CH, HIDDEN // 2), jnp.float32),
        pltpu.VMEM_SHARED((NPAD, HIDDEN // 2), jnp.float32),
        pltpu.SemaphoreType.DMA,
    ],
)
def _agg1_kernel(y_hbm, src_hbm, dst_hbm, zeros_hbm, out_hbm,
                 srcv, dstv, rows, acc, sem):
    cid = lax.axis_index("c")
    sid = lax.axis_index("s")
    plane = y_hbm.at[cid]
    pltpu.sync_copy(src_hbm.at[sid].at[pl.ds(0, TRA)], srcv)
    pltpu.sync_copy(dst_hbm.at[sid].at[pl.ds(0, TRA)], dstv)
    _copy_share(sid, zeros_hbm, acc)
    plsc.subcore_barrier()
    _gs_serial(TRA, plane, srcv, dstv, rows, acc, sem)
    pltpu.sync_copy(src_hbm.at[sid].at[pl.ds(TRA, TRA)], srcv)
    pltpu.sync_copy(dst_hbm.at[sid].at[pl.ds(TRA, TRA)], dstv)
    _gs_serial(TRA, plane, srcv, dstv, rows, acc, sem)
    plsc.subcore_barrier()
    _copy_share(sid, acc, out_hbm.at[cid])


@functools.partial(
    pl.kernel,
    out_type=jax.ShapeDtypeStruct((2, N, OUT), jnp.float32),
    mesh=_sc_mesh(),
    scratch_types=[
        pltpu.VMEM((NCHP, CH), jnp.int32),
        pltpu.VMEM((NCHP, CH), jnp.int32),
        pltpu.VMEM((CH, OUT), jnp.float32),
        pltpu.VMEM_SHARED((NPAD, OUT), jnp.float32),
        pltpu.SemaphoreType.DMA,
    ],
)
def _agg2_kernel(y_hbm, src_hbm, dst_hbm, zeros_hbm, out_hbm,
                 srcv, dstv, rows, acc, sem):
    cid = lax.axis_index("c")
    sid = lax.axis_index("s")
    w = cid * NTILES + sid
    pltpu.sync_copy(src_hbm.at[w], srcv)
    pltpu.sync_copy(dst_hbm.at[w], dstv)
    _copy_share(sid, zeros_hbm, acc)
    plsc.subcore_barrier()
    _gs_serial(NCHP, y_hbm, srcv, dstv, rows, acc, sem)
    plsc.subcore_barrier()
    _copy_share(sid, acc, out_hbm.at[cid])


def _b_body(x_ref, w_ref, deg_ref, y_ref, dinv_ref):
    total = deg_ref[0, :, 0:1] + deg_ref[1, :, 0:1]
    dcol = lax.rsqrt(total + 1.0)
    dinv_ref[...] = dcol
    y = dcol * jnp.dot(x_ref[...], w_ref[...],
                       preferred_element_type=jnp.float32)
    y_ref[0] = y[:, :HIDDEN // 2]
    y_ref[1] = y[:, HIDDEN // 2:]


def _d_body(agg_ref, y1_ref, dinv_ref, b1_ref, w2_ref, y2_ref):
    dinv = dinv_ref[...]
    s = jnp.concatenate([agg_ref[0] + y1_ref[0], agg_ref[1] + y1_ref[1]],
                        axis=1)
    h = jnp.maximum(dinv * s + b1_ref[...], 0.0)
    y2_ref[...] = dinv * jnp.dot(h, w2_ref[...],
                                 preferred_element_type=jnp.float32)


def _f_body(agg_ref, y2_ref, dinv_ref, b2_ref, out_ref):
    s = agg_ref[0] + agg_ref[1] + y2_ref[...]
    out_ref[...] = dinv_ref[...] * s + b2_ref[...]


def _tc_b(x, w1, degp):
    return pl.pallas_call(
        _b_body,
        grid=(N // RB,),
        in_specs=[
            pl.BlockSpec((RB, LATENT), lambda i: (i, 0)),
            pl.BlockSpec((LATENT, HIDDEN), lambda i: (0, 0)),
            pl.BlockSpec((2, RB, DEGW), lambda i: (0, i, 0)),
        ],
        out_specs=[
            pl.BlockSpec((2, RB, HIDDEN // 2), lambda i: (0, i, 0)),
            pl.BlockSpec((RB, 1), lambda i: (i, 0)),
        ],
        out_shape=[
            jax.ShapeDtypeStruct((2, N, HIDDEN // 2), jnp.float32),
            jax.ShapeDtypeStruct((N, 1), jnp.float32),
        ],
    )(x, w1, degp)


def _tc_d(agg1, y1, dinv, b1, w2):
    return pl.pallas_call(
        _d_body,
        grid=(N // RB,),
        in_specs=[
            pl.BlockSpec((2, RB, HIDDEN // 2), lambda i: (0, i, 0)),
            pl.BlockSpec((2, RB, HIDDEN // 2), lambda i: (0, i, 0)),
            pl.BlockSpec((RB, 1), lambda i: (i, 0)),
            pl.BlockSpec((1, HIDDEN), lambda i: (0, 0)),
            pl.BlockSpec((HIDDEN, OUT), lambda i: (0, 0)),
        ],
        out_specs=pl.BlockSpec((RB, OUT), lambda i: (i, 0)),
        out_shape=jax.ShapeDtypeStruct((N, OUT), jnp.float32),
    )(agg1, y1, dinv, b1, w2)


def _tc_f(agg2, y2, dinv, b2):
    return pl.pallas_call(
        _f_body,
        grid=(N // RB,),
        in_specs=[
            pl.BlockSpec((2, RB, OUT), lambda i: (0, i, 0)),
            pl.BlockSpec((RB, OUT), lambda i: (i, 0)),
            pl.BlockSpec((RB, 1), lambda i: (i, 0)),
            pl.BlockSpec((1, OUT), lambda i: (0, 0)),
        ],
        out_specs=pl.BlockSpec((RB, OUT), lambda i: (i, 0)),
        out_shape=jax.ShapeDtypeStruct((N, OUT), jnp.float32),
    )(agg2, y2, dinv, b2)


def kernel(x, edge_index, W1, b1, W2, b2):
    src = edge_index[0].astype(jnp.int32)
    dst = edge_index[1].astype(jnp.int32)
    pad = EP - E
    pad_src = jnp.arange(pad, dtype=jnp.int32) % N
    pad_dst = N + jnp.arange(pad, dtype=jnp.int32) % 8
    srcf = jnp.concatenate([src, pad_src])
    dstf = jnp.concatenate([dst, pad_dst])
    srcp = srcf.reshape(32, NCHP, CH)
    dstp = dstf.reshape(32, NCHP, CH)
    src1 = srcf.reshape(NTILES, 2 * TRA, CH)
    dst1 = dstf.reshape(NTILES, 2 * TRA, CH)

    zeros_h = jnp.zeros((N, HIDDEN // 2), jnp.float32)
    zeros_o = jnp.zeros((N, OUT), jnp.float32)
    zeros_d = jnp.zeros((N, DEGW), jnp.float32)
    ones_d = jnp.ones((CH, DEGW), jnp.float32)

    degp = _deg_kernel(dstp, ones_d, zeros_d)
    y1, dinv = _tc_b(x, W1, degp)
    agg1 = _agg1_kernel(y1, src1, dst1, zeros_h)
    y2 = _tc_d(agg1, y1, dinv, b1.reshape(1, HIDDEN), W2)
    agg2 = _agg2_kernel(y2, srcp, dstp, zeros_o)
    return _tc_f(agg2, y2, dinv, b2.reshape(1, OUT))
